# baseline (speedup 1.0000x reference)
import functools

import jax
import jax.numpy as jnp
import numpy as np
from jax import lax
from jax.experimental import pallas as pl
from jax.experimental.pallas import tpu as pltpu
from jax.experimental.pallas import tpu_sc as plsc

_D = 128
_N_TILES = 16
_N_CORES = 2
_CHUNK = 128
_WIN = 240


def _emb_body(x_ref, w_ref, b_ref, o_ref):
    o_ref[...] = (
        jnp.dot(x_ref[...].astype(jnp.bfloat16), w_ref[...].astype(jnp.bfloat16),
                preferred_element_type=jnp.float32)
        + b_ref[...]
    )


def _emb(x, w, b):
    return pl.pallas_call(
        _emb_body,
        out_shape=jax.ShapeDtypeStruct((x.shape[0], w.shape[1]), jnp.float32),
    )(x, w, b.reshape(1, -1))


@functools.cache
def _make_gather(n_tiles_slot, s_slot):
    n_chunks = s_slot // _CHUNK
    mesh = plsc.VectorSubcoreMesh(core_axis_name="c", subcore_axis_name="s")

    @functools.partial(
        pl.kernel,
        mesh=mesh,
        out_type=jax.ShapeDtypeStruct((n_tiles_slot * s_slot, _D), jnp.float32),
        scratch_types=[
            pltpu.VMEM((_CHUNK,), jnp.int32),
            pltpu.VMEM((_CHUNK, _D), jnp.float32),
            pltpu.SemaphoreType.DMA,
        ],
    )
    def gat(h_hbm, idx_hbm, out_hbm, idx_v, rows_v, sem):
        c = lax.axis_index("c")
        s = lax.axis_index("s")
        wid = c * _N_TILES + s

        def body(j, carry):
            base = j * _CHUNK
            pltpu.sync_copy(idx_hbm.at[wid, pl.ds(base, _CHUNK)], idx_v)
            pltpu.async_copy(h_hbm.at[idx_v], rows_v, sem).wait()
            pltpu.sync_copy(rows_v,
                            out_hbm.at[pl.ds(wid * s_slot + base, _CHUNK)])
            return carry

        lax.fori_loop(0, n_chunks, body, 0)

    return gat


def _prep_edges(edge_index):
    e = edge_index.shape[1]
    src, dst = edge_index[0], edge_index[1]
    perm = jnp.argsort(dst, stable=True)
    ds = dst[perm].astype(jnp.int32)
    sp = src[perm].astype(jnp.int32)
    nw = _N_CORES * _N_TILES
    s_slot = -(-e // (nw * _CHUNK)) * _CHUNK
    pad = nw * s_slot - e
    sp_rep = jnp.concatenate([sp, jnp.zeros((pad,), jnp.int32)]).reshape(nw, s_slot)
    return sp_rep, ds, s_slot


def _segsum(h, prep, n):
    sp_rep, ds, s_slot = prep
    e = ds.shape[0]
    updates = _make_gather(_N_CORES * _N_TILES, s_slot)(h, sp_rep)[:e]
    return jax.ops.segment_sum(updates, ds, num_segments=n)


def _bn(x, g, b):
    mu = jnp.mean(x, axis=0, keepdims=True)
    var = jnp.var(x, axis=0, keepdims=True)
    return (x - mu) / jnp.sqrt(var + 1e-5) * g + b


def _mlp_fwd(layers, x):
    n = len(layers)
    for i, l in enumerate(layers):
        x = x @ l['W'] + l['b']
        if i < n - 1:
            x = jax.nn.relu(_bn(x, l['gamma'], l['beta']))
    return x


def _gin_fwd(p, h, agg):
    h2 = (1.0 + p['eps']) * h + agg
    h2 = _mlp_fwd(p['mlp'], h2)
    h2 = jax.nn.relu(_bn(h2, p['bn_apply_g'], p['bn_apply_b']))
    h2 = _bn(h2, p['bn_layer_g'], p['bn_layer_b'])
    return h + h2


def kernel(x, edge_index, x_p, edge_index_p, extension_matrix, params):
    p = params
    n_g, n_p = x.shape[0], x_p.shape[0]
    prep_g = _prep_edges(edge_index)
    prep_p = _prep_edges(edge_index_p)

    h = _emb(x, p['emb_W'], p['emb_b'])
    for gp in p['gin_g']:
        h = _gin_fwd(gp, h, _segsum(h, prep_g, n_g))
    h = _mlp_fwd(p['mlp_g'], h)

    hp = _emb(x_p, p['emb_W'], p['emb_b'])
    for gp in p['gin_p']:
        hp = _gin_fwd(gp, hp, _segsum(hp, prep_p, n_p))
    hp = _mlp_fwd(p['mlp_p'], hp)

    p_rep = jnp.mean(hp, axis=0, keepdims=True)
    ext = extension_matrix @ p_rep
    hc = jnp.concatenate([h, ext], axis=1)
    return jax.nn.sigmoid(_mlp_fwd(p['predict'], hc))
